# baseline (speedup 1.0000x reference)
import functools
import numpy as np
from math import comb, factorial, floor

import jax
import jax.numpy as jnp
from jax import lax
from jax.experimental import pallas as pl
from jax.experimental.pallas import tpu as pltpu
from jax.experimental.pallas import tpu_sc as plsc

_MAX_L = 4


def _tables(max_l):
    dst, pows, cs, ns = [], [], [], []
    for l in range(max_l + 1):
        for m in range(-l, l + 1):
            am = abs(m)
            n_lm = (1.0 / (2.0 ** am * factorial(l))) * np.sqrt(
                2.0 * factorial(l + am) * factorial(l - am) / (2.0 if m == 0 else 1.0))
            ns.append(n_lm)
            vm = 0.5 if m < 0 else 0.0
            vmax = floor(am / 2.0 - vm) + vm
            for t in range(0, (l - am) // 2 + 1):
                for u in range(0, t + 1):
                    v = vm
                    while v <= vmax + 1e-9:
                        c = ((-1.0) ** int(round(t + v - vm))) * (0.25 ** t) \
                            * comb(l, t) * comb(l - t, am + t) * comb(t, u) * comb(am, int(round(2 * v)))
                        dst.append(l * (l + 1) + m)
                        pows.append([int(round(2 * t + am - 2 * (u + v))),
                                     int(round(2 * (u + v))),
                                     int(l - 2 * t - am)])
                        cs.append(c)
                        v += 1.0
    return dst, pows, cs, ns


def _channel_terms():
    dst, pows, cs, ns = _tables(_MAX_L)
    n_out = len(ns)
    terms = {}
    for d, p, c in zip(dst, pows, cs):
        key = (d, tuple(p))
        terms[key] = terms.get(key, 0.0) + c
    chans = [[] for _ in range(n_out)]
    for (d, p), c in terms.items():
        chans[d].append((float(c) * float(ns[d]), p))
    return chans


_CHANS = _channel_terms()
_N_OUT = len(_CHANS)

_C = 1280
_NW = 32
_LANES = 16
_B_IN = 3 * _C
_B_OUT = _N_OUT * _C


def _eval_channels(x, y, z, o_ref, obase, off):
    xp = [None, x, x * x, None, None]
    yp = [None, y, y * y, None, None]
    zp = [None, z, z * z, None, None]
    xp[3], xp[4] = xp[2] * x, xp[2] * xp[2]
    yp[3], yp[4] = yp[2] * y, yp[2] * yp[2]
    zp[3], zp[4] = zp[2] * z, zp[2] * zp[2]
    pw = (xp, yp, zp)
    mono = {}
    for terms in _CHANS:
        for _, p in terms:
            if p in mono:
                continue
            m = None
            for pwc, e in zip(pw, p):
                if e:
                    m = pwc[e] if m is None else m * pwc[e]
            mono[p] = m
    for j, terms in enumerate(_CHANS):
        acc = None
        for coef, p in terms:
            m = mono[p]
            t = jnp.full_like(x, coef) if m is None else m * coef
            acc = t if acc is None else acc + t
        o_ref[pl.ds(obase + j * _C + off, _LANES)] = acc


def kernel(xyz):
    in_shape = xyz.shape
    rows = xyz.reshape(-1, 3)
    n = rows.shape[0]
    xt = rows.T.reshape(-1)
    nchunks = n // _C
    rounds = (nchunks + _NW - 1) // _NW
    rounds += rounds % 2

    mesh = plsc.VectorSubcoreMesh(core_axis_name="c", subcore_axis_name="s")

    @functools.partial(
        pl.kernel,
        mesh=mesh,
        out_type=jax.ShapeDtypeStruct((_N_OUT * n,), jnp.float32),
        scratch_types=[
            pltpu.VMEM((2 * _B_IN,), jnp.float32),
            pltpu.VMEM((2 * _B_OUT,), jnp.float32),
            pltpu.SemaphoreType.DMA,
            pltpu.SemaphoreType.DMA,
            pltpu.SemaphoreType.DMA,
            pltpu.SemaphoreType.DMA,
        ],
    )
    def sc_kernel(xt_hbm, out_hbm, xin, o, si0, si1, so0, so1):
        wid = lax.axis_index("s") * 2 + lax.axis_index("c")

        def valid(r):
            return r * _NW + wid < nchunks

        def base_of(r):
            return (r * _NW + wid) * _C

        def issue_in(r, b, sem):
            base = base_of(r)
            for i in range(3):
                pltpu.async_copy(
                    xt_hbm.at[pl.ds(i * n + base, _C)],
                    xin.at[pl.ds(b * _B_IN + i * _C, _C)], sem)

        def wait_in(b, sem):
            pltpu.make_async_copy(xt_hbm.at[pl.ds(0, _B_IN)],
                                  xin.at[pl.ds(b * _B_IN, _B_IN)], sem).wait()

        def fire_out(r, b, sem):
            base = base_of(r)
            for j in range(_N_OUT):
                pltpu.async_copy(
                    o.at[pl.ds(b * _B_OUT + j * _C, _C)],
                    out_hbm.at[pl.ds(j * n + base, _C)], sem)

        def wait_out(b, sem):
            pltpu.make_async_copy(out_hbm.at[pl.ds(0, _B_OUT)],
                                  o.at[pl.ds(b * _B_OUT, _B_OUT)], sem).wait()

        def compute(b):
            @plsc.parallel_loop(0, _C, step=_LANES, unroll=4)
            def vec_body(off):
                x = xin[pl.ds(b * _B_IN + off, _LANES)]
                y = xin[pl.ds(b * _B_IN + _C + off, _LANES)]
                z = xin[pl.ds(b * _B_IN + 2 * _C + off, _LANES)]
                _eval_channels(x, y, z, o, b * _B_OUT, off)

        @pl.when(valid(0))
        def _():
            issue_in(0, 0, si0)

        @pl.when(valid(1))
        def _():
            issue_in(1, 1, si1)

        def dbl_body(d, carry):
            r0 = 2 * d
            for (b, sem_i, sem_o, r) in ((0, si0, so0, r0), (1, si1, so1, r0 + 1)):
                @pl.when(valid(r))
                def _():
                    wait_in(b, sem_i)

                    @pl.when(d > 0)
                    def _():
                        wait_out(b, sem_o)

                    compute(b)
                    fire_out(r, b, sem_o)

                    @pl.when(valid(r + 2))
                    def _():
                        issue_in(r + 2, b, sem_i)

            return carry

        lax.fori_loop(0, rounds // 2, dbl_body, 0)

        @pl.when(valid(0))
        def _():
            wait_out(0, so0)

        @pl.when(valid(1))
        def _():
            wait_out(1, so1)

    out = sc_kernel(xt)
    return out.reshape(_N_OUT, n).T.reshape(*in_shape[:-1], _N_OUT)


# speedup vs baseline: 33.5527x; 33.5527x over previous
import functools
import numpy as np
from math import comb, factorial, floor

import jax
import jax.numpy as jnp
from jax import lax
from jax.experimental import pallas as pl
from jax.experimental.pallas import tpu as pltpu
from jax.experimental.pallas import tpu_sc as plsc

_MAX_L = 4


def _tables(max_l):
    dst, pows, cs, ns = [], [], [], []
    for l in range(max_l + 1):
        for m in range(-l, l + 1):
            am = abs(m)
            n_lm = (1.0 / (2.0 ** am * factorial(l))) * np.sqrt(
                2.0 * factorial(l + am) * factorial(l - am) / (2.0 if m == 0 else 1.0))
            ns.append(n_lm)
            vm = 0.5 if m < 0 else 0.0
            vmax = floor(am / 2.0 - vm) + vm
            for t in range(0, (l - am) // 2 + 1):
                for u in range(0, t + 1):
                    v = vm
                    while v <= vmax + 1e-9:
                        c = ((-1.0) ** int(round(t + v - vm))) * (0.25 ** t) \
                            * comb(l, t) * comb(l - t, am + t) * comb(t, u) * comb(am, int(round(2 * v)))
                        dst.append(l * (l + 1) + m)
                        pows.append([int(round(2 * t + am - 2 * (u + v))),
                                     int(round(2 * (u + v))),
                                     int(l - 2 * t - am)])
                        cs.append(c)
                        v += 1.0
    return dst, pows, cs, ns


def _channel_terms():
    dst, pows, cs, ns = _tables(_MAX_L)
    n_out = len(ns)
    terms = {}
    for d, p, c in zip(dst, pows, cs):
        key = (d, tuple(p))
        terms[key] = terms.get(key, 0.0) + c
    chans = [[] for _ in range(n_out)]
    for (d, p), c in terms.items():
        chans[d].append((float(c) * float(ns[d]), p))
    return chans


_CHANS = _channel_terms()
_N_OUT = len(_CHANS)

_C = 1280
_NW = 32
_LANES = 16


def _eval_channels(x, y, z, o_ref, b, off):
    xp = [None, x, x * x, None, None]
    yp = [None, y, y * y, None, None]
    zp = [None, z, z * z, None, None]
    xp[3], xp[4] = xp[2] * x, xp[2] * xp[2]
    yp[3], yp[4] = yp[2] * y, yp[2] * yp[2]
    zp[3], zp[4] = zp[2] * z, zp[2] * zp[2]
    pw = (xp, yp, zp)
    mono = {}
    for terms in _CHANS:
        for _, p in terms:
            if p in mono:
                continue
            m = None
            for pwc, e in zip(pw, p):
                if e:
                    m = pwc[e] if m is None else m * pwc[e]
            mono[p] = m
    for j, terms in enumerate(_CHANS):
        acc = None
        for coef, p in terms:
            m = mono[p]
            t = jnp.full_like(x, coef) if m is None else m * coef
            acc = t if acc is None else acc + t
        o_ref[b, j, pl.ds(off, _LANES)] = acc


def kernel(xyz):
    in_shape = xyz.shape
    rows = xyz.reshape(-1, 3)
    n = rows.shape[0]
    xt = rows.T
    nchunks = n // _C
    rounds = (nchunks + _NW - 1) // _NW
    rounds += rounds % 2

    mesh = plsc.VectorSubcoreMesh(core_axis_name="c", subcore_axis_name="s")

    @functools.partial(
        pl.kernel,
        mesh=mesh,
        out_type=jax.ShapeDtypeStruct((_N_OUT, n), jnp.float32),
        scratch_types=[
            pltpu.VMEM((2, 3, _C), jnp.float32),
            pltpu.VMEM((2, _N_OUT, _C), jnp.float32),
            pltpu.SemaphoreType.DMA,
            pltpu.SemaphoreType.DMA,
            pltpu.SemaphoreType.DMA,
            pltpu.SemaphoreType.DMA,
        ],
    )
    def sc_kernel(xt_hbm, out_hbm, xin, o, si0, si1, so0, so1):
        wid = lax.axis_index("s") * 2 + lax.axis_index("c")

        def valid(r):
            return r * _NW + wid < nchunks

        def base_of(r):
            return (r * _NW + wid) * _C

        def issue_in(r, b, sem):
            base = base_of(r)
            pltpu.async_copy(xt_hbm.at[:, pl.ds(base, _C)], xin.at[b], sem)

        def wait_in(b, sem):
            pltpu.make_async_copy(xt_hbm.at[:, pl.ds(0, _C)],
                                  xin.at[b], sem).wait()

        def fire_out(r, b, sem):
            base = base_of(r)
            pltpu.async_copy(o.at[b], out_hbm.at[:, pl.ds(base, _C)], sem)

        def wait_out(b, sem):
            pltpu.make_async_copy(out_hbm.at[:, pl.ds(0, _C)],
                                  o.at[b], sem).wait()

        def compute(b):
            @plsc.parallel_loop(0, _C, step=_LANES, unroll=4)
            def vec_body(off):
                x = xin[b, 0, pl.ds(off, _LANES)]
                y = xin[b, 1, pl.ds(off, _LANES)]
                z = xin[b, 2, pl.ds(off, _LANES)]
                _eval_channels(x, y, z, o, b, off)

        @pl.when(valid(0))
        def _():
            issue_in(0, 0, si0)

        @pl.when(valid(1))
        def _():
            issue_in(1, 1, si1)

        def dbl_body(d, carry):
            r0 = 2 * d
            for (b, sem_i, sem_o, r) in ((0, si0, so0, r0), (1, si1, so1, r0 + 1)):
                @pl.when(valid(r))
                def _():
                    wait_in(b, sem_i)

                    @pl.when(d > 0)
                    def _():
                        wait_out(b, sem_o)

                    compute(b)
                    fire_out(r, b, sem_o)

                    @pl.when(valid(r + 2))
                    def _():
                        issue_in(r + 2, b, sem_i)

            return carry

        lax.fori_loop(0, rounds // 2, dbl_body, 0)

        @pl.when(valid(0))
        def _():
            wait_out(0, so0)

        @pl.when(valid(1))
        def _():
            wait_out(1, so1)

    out = sc_kernel(xt)
    return out.reshape(_N_OUT, n).T.reshape(*in_shape[:-1], _N_OUT)
